# baseline (speedup 1.0000x reference)
import functools

import jax
import jax.numpy as jnp
from jax import lax
from jax.experimental import pallas as pl
import jax.experimental.pallas.tpu as pltpu
from jax.experimental.pallas import tpu_sc as plsc

N_W = 100000
N_GRAPHS = 512
D_IN = 128
D_OUT = 32
D_AUG = 48

NC = 2
NS = 16
NW = NC * NS
P = 100
SUB = 1000
NSTREAM = SUB // P
NCHUNK = N_W // SUB
NQ = 4
QROWS = N_W // NQ
QCHUNK = NCHUNK // NQ
TPQ = NW // NQ
KMAX = -(-QCHUNK // TPQ)


def _proj_body(x0_ref, x1_ref, x2_ref, x3_ref, w_ref,
               y0_ref, y1_ref, y2_ref, y3_ref):
    w = w_ref[...]
    lane = jax.lax.broadcasted_iota(jnp.int32, (1, D_AUG), 1)
    ones_col = jnp.where(lane == D_OUT, 1.0, 0.0)
    for x_ref, y_ref in ((x0_ref, y0_ref), (x1_ref, y1_ref),
                         (x2_ref, y2_ref), (x3_ref, y3_ref)):
        x = jnp.maximum(x_ref[...], 0.0)
        y_ref[...] = jax.lax.dot_general(
            x, w, (((1,), (0,)), ((), ())),
            preferred_element_type=jnp.float32) + ones_col


def _proj(x, w_aug):
    pblk = 1000
    nstep = QROWS // pblk

    def xspec(j):
        return pl.BlockSpec((pblk, D_IN),
                            lambda i, j=j: (j * nstep + i, 0))

    yspec = pl.BlockSpec((pblk, D_AUG), lambda i: (i, 0))

    return pl.pallas_call(
        _proj_body,
        grid=(nstep,),
        in_specs=[
            xspec(0), xspec(1), xspec(2), xspec(3),
            pl.BlockSpec((D_IN, D_AUG), lambda i: (0, 0)),
        ],
        out_specs=[yspec, yspec, yspec, yspec],
        out_shape=[jax.ShapeDtypeStruct((QROWS, D_AUG), jnp.float32)] * NQ,
    )(x, x, x, x, w_aug)


def _sc_segsum(ys, batch3, zeros_acc):
    mesh = plsc.VectorSubcoreMesh(core_axis_name="c", subcore_axis_name="s")

    @functools.partial(
        pl.kernel,
        mesh=mesh,
        compiler_params=pltpu.CompilerParams(use_tc_tiling_on_sc=False),
        out_type=jax.ShapeDtypeStruct((NC, N_GRAPHS, D_AUG), jnp.float32),
        scratch_types=[
            pltpu.VMEM((NSTREAM, P), jnp.int32),
            pltpu.VMEM((SUB, D_AUG), jnp.float32),
            pltpu.VMEM_SHARED((N_GRAPHS, D_AUG), jnp.float32),
            pltpu.SemaphoreType.DMA,
        ],
    )
    def body(y0_hbm, y1_hbm, y2_hbm, y3_hbm, b_hbm, zacc_hbm,
             sums_hbm, idx_v, rows_v, acc_sh, sem):
        c = lax.axis_index("c")
        s = lax.axis_index("s")
        wid = c * NS + s

        @pl.when(s == 0)
        def _init():
            pltpu.sync_copy(zacc_hbm, acc_sh)

        plsc.subcore_barrier()

        yqs = (y0_hbm, y1_hbm, y2_hbm, y3_hbm)
        for q in range(NQ):

            @pl.when(wid // TPQ == q)
            def _quarter(q=q):
                t = wid % TPQ
                for k in range(KMAX):
                    g_loc = t + k * TPQ

                    @pl.when(g_loc < QCHUNK)
                    def _chunk(g_loc=g_loc, q=q):
                        pltpu.sync_copy(b_hbm.at[q * QCHUNK + g_loc], idx_v)
                        pltpu.sync_copy(
                            yqs[q].at[pl.ds(g_loc * SUB, SUB)], rows_v)
                        copies = []
                        for j in range(NSTREAM):
                            copies.append(pltpu.async_copy(
                                rows_v.at[pl.ds(j * P, P)],
                                acc_sh.at[idx_v.at[j]], sem, add=True))
                        for cp in copies:
                            cp.wait()

        plsc.subcore_barrier()

        @pl.when(s == 0)
        def _flush():
            pltpu.sync_copy(acc_sh, sums_hbm.at[c])

    return body(*ys, batch3, zeros_acc)


def _finish_body(s_ref, bias_ref, out_ref):
    sums = s_ref[0] + s_ref[1]
    cnt = jnp.maximum(sums[:, D_OUT:D_OUT + 1], 1.0)
    out_ref[...] = sums[:, :D_OUT] / cnt + bias_ref[...]


def _finish(sums, fc_b):
    return pl.pallas_call(
        _finish_body,
        in_specs=[
            pl.BlockSpec((NC, N_GRAPHS, D_AUG), lambda: (0, 0, 0)),
            pl.BlockSpec((1, D_OUT), lambda: (0, 0)),
        ],
        out_specs=pl.BlockSpec((N_GRAPHS, D_OUT), lambda: (0, 0)),
        out_shape=jax.ShapeDtypeStruct((N_GRAPHS, D_OUT), jnp.float32),
    )(sums, fc_b.reshape(1, D_OUT))


@jax.jit
def _pool_fc(x_workload, workload_batch, fc_W, fc_b):
    w_aug = jnp.zeros((D_IN, D_AUG), jnp.float32).at[:, :D_OUT].set(fc_W)
    ys = _proj(x_workload, w_aug)
    batch3 = workload_batch.reshape(NCHUNK, NSTREAM, P)
    zeros_acc = jnp.zeros((N_GRAPHS, D_AUG), jnp.float32)
    sums = _sc_segsum(ys, batch3, zeros_acc)
    return _finish(sums, fc_b)


def kernel(x_workload, x_vm, x_host, edge_index_assigned, edge_index_runs,
           workload_batch, conv1_gcn_W, conv1_gcn_b, conv1_sage_Wl,
           conv1_sage_Wr, conv1_sage_b, conv2_gcn_W, conv2_gcn_b,
           conv2_sage_Wl, conv2_sage_Wr, conv2_sage_b, fc_W, fc_b):
    return _pool_fc(x_workload, workload_batch, fc_W, fc_b)
